# baseline (speedup 1.0000x reference)
import functools

import jax
import jax.numpy as jnp
from jax import lax
from jax.experimental import pallas as pl
from jax.experimental.pallas import tpu as pltpu
from jax.experimental.pallas import tpu_sc as plsc

_R = 32
_B = 8
_C = 64
_N = 65536
_V = _R * _R * _R
_P = 8192


def _coords_body(coords_ref, norm_ref, ids_ref):
    c = coords_ref[...]
    mean = jnp.mean(c, axis=2, keepdims=True)
    n = (c - mean + 1.0) / 2.0
    scaled = jnp.clip(n * float(_R), 0.0, float(_R - 1))
    norm_ref[...] = scaled
    v = jnp.round(scaled).astype(jnp.int32)
    ids_ref[...] = (v[:, 0:1, :] * (_R * _R) + v[:, 1:2, :] * _R
                    + v[:, 2:3, :])


def _coords_call(coords):
    return pl.pallas_call(
        _coords_body,
        grid=(_B,),
        in_specs=[pl.BlockSpec((1, 3, _N), lambda b: (b, 0, 0))],
        out_specs=[
            pl.BlockSpec((1, 3, _N), lambda b: (b, 0, 0)),
            pl.BlockSpec((1, 1, _N), lambda b: (b, 0, 0)),
        ],
        out_shape=[
            jax.ShapeDtypeStruct((_B, 3, _N), jnp.float32),
            jax.ShapeDtypeStruct((_B, 1, _N), jnp.int32),
        ],
    )(coords)


def _sc_body(feat_hbm, ids_hbm, out_hbm, inv_hbm,
             ids_v, acc_v, buf0, buf1, sem0, sem1):
    cid = lax.axis_index("c")
    sid = lax.axis_index("s")
    wid = cid * 16 + sid
    b = wid // 4
    q = wid % 4
    c0 = q * 16
    zero16 = jnp.zeros((16,), jnp.float32)
    one16 = jnp.full((16,), 1.0, jnp.float32)

    pltpu.sync_copy(ids_hbm.at[b], ids_v)

    def zero_acc(j, _):
        base = j * 64
        for u in range(4):
            acc_v[pl.ds(base + u * 16, 16)] = zero16
        return 0

    lax.fori_loop(0, _V // 64, zero_acc, 0)

    def count(i, _):
        base = i * 64
        for u in range(4):
            plsc.addupdate_scatter(
                acc_v, [ids_v[pl.ds(base + u * 16, 16)]], one16)
        return 0
    lax.fori_loop(0, _N // 64, count, 0)

    def to_inv(j, _):
        s = pl.ds(j * 16, 16)
        acc_v[s] = 1.0 / jnp.maximum(acc_v[s], 1.0)
        return 0
    lax.fori_loop(0, _V // 16, to_inv, 0)
    pltpu.sync_copy(acc_v, inv_hbm.at[b, q])

    def scatter_chunk(buf, base):
        def inner(i, _):
            p = base + i * 64
            qq = i * 64
            for u in range(4):
                plsc.addupdate_scatter(
                    acc_v, [ids_v[pl.ds(p + u * 16, 16)]],
                    buf[pl.ds(qq + u * 16, 16)])
            return 0
        lax.fori_loop(0, _P // 64, inner, 0)

    def chan(ci, _):
        c = c0 + ci
        lax.fori_loop(0, _V // 64, zero_acc, 0)
        pltpu.async_copy(feat_hbm.at[b, c, pl.ds(0, _P)], buf0, sem0)

        def pair(k2, _):
            base = k2 * (2 * _P)
            pltpu.async_copy(
                feat_hbm.at[b, c, pl.ds(base + _P, _P)], buf1, sem1)
            pltpu.make_async_copy(
                feat_hbm.at[b, c, pl.ds(0, _P)], buf0, sem0).wait()
            scatter_chunk(buf0, base)

            @pl.when(k2 < _N // (2 * _P) - 1)
            def _():
                pltpu.async_copy(
                    feat_hbm.at[b, c, pl.ds(base + 2 * _P, _P)], buf0, sem0)
            pltpu.make_async_copy(
                feat_hbm.at[b, c, pl.ds(0, _P)], buf1, sem1).wait()
            scatter_chunk(buf1, base + _P)
            return 0
        lax.fori_loop(0, _N // (2 * _P), pair, 0)

        pltpu.async_copy(inv_hbm.at[b, q, pl.ds(0, _P)], buf0, sem0)

        def scale_pair(j2, _):
            base = j2 * (2 * _P)
            pltpu.async_copy(
                inv_hbm.at[b, q, pl.ds(base + _P, _P)], buf1, sem1)
            pltpu.make_async_copy(
                inv_hbm.at[b, q, pl.ds(0, _P)], buf0, sem0).wait()

            def mul0(i, _):
                for u in range(4):
                    s = pl.ds(base + i * 64 + u * 16, 16)
                    acc_v[s] = acc_v[s] * buf0[pl.ds(i * 64 + u * 16, 16)]
                return 0
            lax.fori_loop(0, _P // 64, mul0, 0)

            @pl.when(j2 < _V // (2 * _P) - 1)
            def _():
                pltpu.async_copy(
                    inv_hbm.at[b, q, pl.ds(base + 2 * _P, _P)], buf0, sem0)
            pltpu.make_async_copy(
                inv_hbm.at[b, q, pl.ds(0, _P)], buf1, sem1).wait()

            def mul1(i, _):
                for u in range(4):
                    s = pl.ds(base + _P + i * 64 + u * 16, 16)
                    acc_v[s] = acc_v[s] * buf1[pl.ds(i * 64 + u * 16, 16)]
                return 0
            lax.fori_loop(0, _P // 64, mul1, 0)
            return 0
        lax.fori_loop(0, _V // (2 * _P), scale_pair, 0)
        pltpu.sync_copy(acc_v, out_hbm.at[b, c])
        return 0
    lax.fori_loop(0, 16, chan, 0)


def _sc_call(features, ids):
    mesh = plsc.VectorSubcoreMesh(core_axis_name="c", subcore_axis_name="s")
    f = functools.partial(
        pl.kernel,
        out_type=[
            jax.ShapeDtypeStruct((_B, _C, _V), jnp.float32),
            jax.ShapeDtypeStruct((_B, 4, _V), jnp.float32),
        ],
        mesh=mesh,
        compiler_params=pltpu.CompilerParams(needs_layout_passes=False),
        scratch_types=[
            pltpu.VMEM((_N,), jnp.int32),
            pltpu.VMEM((_V,), jnp.float32),
            pltpu.VMEM((_P,), jnp.float32),
            pltpu.VMEM((_P,), jnp.float32),
            pltpu.SemaphoreType.DMA,
            pltpu.SemaphoreType.DMA,
        ],
    )(_sc_body)
    return f(features, ids)


def kernel(features, coords):
    coords = lax.stop_gradient(coords)
    norm, ids3 = _coords_call(coords)
    ids = ids3.reshape(_B, _N)
    vox, _ = _sc_call(features, ids)
    return vox.reshape(_B, _C, _R, _R, _R), norm


# speedup vs baseline: 1.8711x; 1.8711x over previous
import functools

import jax
import jax.numpy as jnp
from jax import lax
from jax.experimental import pallas as pl
from jax.experimental.pallas import tpu as pltpu
from jax.experimental.pallas import tpu_sc as plsc

_R = 32
_B = 8
_C = 64
_N = 65536
_V = _R * _R * _R
_P = 8192


def _coords_body(coords_ref, norm_ref, ids_ref):
    c = coords_ref[...]
    mean = jnp.mean(c, axis=2, keepdims=True)
    n = (c - mean + 1.0) / 2.0
    scaled = jnp.clip(n * float(_R), 0.0, float(_R - 1))
    norm_ref[...] = scaled
    v = jnp.round(scaled).astype(jnp.int32)
    ids_ref[...] = (v[:, 0:1, :] * (_R * _R) + v[:, 1:2, :] * _R
                    + v[:, 2:3, :])


def _coords_call(coords):
    return pl.pallas_call(
        _coords_body,
        grid=(_B,),
        in_specs=[pl.BlockSpec((1, 3, _N), lambda b: (b, 0, 0))],
        out_specs=[
            pl.BlockSpec((1, 3, _N), lambda b: (b, 0, 0)),
            pl.BlockSpec((1, 1, _N), lambda b: (b, 0, 0)),
        ],
        out_shape=[
            jax.ShapeDtypeStruct((_B, 3, _N), jnp.float32),
            jax.ShapeDtypeStruct((_B, 1, _N), jnp.int32),
        ],
    )(coords)


def _sc_body(feat_hbm, ids_hbm, out_hbm, inv_hbm,
             ids_v, acc_v, buf0, buf1, sem0, sem1):
    cid = lax.axis_index("c")
    sid = lax.axis_index("s")
    wid = cid * 16 + sid
    b = wid // 4
    q = wid % 4
    c0 = q * 16
    zero16 = jnp.zeros((16,), jnp.float32)
    one16 = jnp.full((16,), 1.0, jnp.float32)

    pltpu.sync_copy(ids_hbm.at[b], ids_v)

    def zero_all_acc():
        @plsc.parallel_loop(0, _V // 16, 1, unroll=8)
        def _(j):
            acc_v[pl.ds(j * 16, 16)] = zero16

    zero_all_acc()

    @plsc.parallel_loop(0, _N // 16, 1, unroll=8)
    def _(i):
        plsc.addupdate_scatter(acc_v, [ids_v[pl.ds(i * 16, 16)]], one16)

    @plsc.parallel_loop(0, _V // 16, 1, unroll=4)
    def _(j):
        s = pl.ds(j * 16, 16)
        acc_v[s] = 1.0 / jnp.maximum(acc_v[s], 1.0)

    pltpu.sync_copy(acc_v, inv_hbm.at[b, q])

    def scatter_chunk(buf, base):
        @plsc.parallel_loop(0, _P // 16, 1, unroll=8)
        def _(i):
            plsc.addupdate_scatter(
                acc_v, [ids_v[pl.ds(base + i * 16, 16)]],
                buf[pl.ds(i * 16, 16)])

    def chan(ci, _):
        c = c0 + ci
        zero_all_acc()
        pltpu.async_copy(feat_hbm.at[b, c, pl.ds(0, _P)], buf0, sem0)

        def pair(k2, _):
            base = k2 * (2 * _P)
            pltpu.async_copy(
                feat_hbm.at[b, c, pl.ds(base + _P, _P)], buf1, sem1)
            pltpu.make_async_copy(
                feat_hbm.at[b, c, pl.ds(0, _P)], buf0, sem0).wait()
            scatter_chunk(buf0, base)

            @pl.when(k2 < _N // (2 * _P) - 1)
            def _():
                pltpu.async_copy(
                    feat_hbm.at[b, c, pl.ds(base + 2 * _P, _P)], buf0, sem0)
            pltpu.make_async_copy(
                feat_hbm.at[b, c, pl.ds(0, _P)], buf1, sem1).wait()
            scatter_chunk(buf1, base + _P)
            return 0
        lax.fori_loop(0, _N // (2 * _P), pair, 0)

        pltpu.async_copy(inv_hbm.at[b, q, pl.ds(0, _P)], buf0, sem0)

        def scale_pair(j2, _):
            base = j2 * (2 * _P)
            pltpu.async_copy(
                inv_hbm.at[b, q, pl.ds(base + _P, _P)], buf1, sem1)
            pltpu.make_async_copy(
                inv_hbm.at[b, q, pl.ds(0, _P)], buf0, sem0).wait()

            @plsc.parallel_loop(0, _P // 16, 1, unroll=8)
            def _(i):
                s = pl.ds(base + i * 16, 16)
                acc_v[s] = acc_v[s] * buf0[pl.ds(i * 16, 16)]

            @pl.when(j2 < _V // (2 * _P) - 1)
            def _():
                pltpu.async_copy(
                    inv_hbm.at[b, q, pl.ds(base + 2 * _P, _P)], buf0, sem0)
            pltpu.make_async_copy(
                inv_hbm.at[b, q, pl.ds(0, _P)], buf1, sem1).wait()

            @plsc.parallel_loop(0, _P // 16, 1, unroll=8)
            def _(i):
                s = pl.ds(base + _P + i * 16, 16)
                acc_v[s] = acc_v[s] * buf1[pl.ds(i * 16, 16)]
            return 0
        lax.fori_loop(0, _V // (2 * _P), scale_pair, 0)
        pltpu.sync_copy(acc_v, out_hbm.at[b, c])
        return 0
    lax.fori_loop(0, 16, chan, 0)


def _sc_call(features, ids):
    mesh = plsc.VectorSubcoreMesh(core_axis_name="c", subcore_axis_name="s")
    f = functools.partial(
        pl.kernel,
        out_type=[
            jax.ShapeDtypeStruct((_B, _C, _V), jnp.float32),
            jax.ShapeDtypeStruct((_B, 4, _V), jnp.float32),
        ],
        mesh=mesh,
        compiler_params=pltpu.CompilerParams(needs_layout_passes=False),
        scratch_types=[
            pltpu.VMEM((_N,), jnp.int32),
            pltpu.VMEM((_V,), jnp.float32),
            pltpu.VMEM((_P,), jnp.float32),
            pltpu.VMEM((_P,), jnp.float32),
            pltpu.SemaphoreType.DMA,
            pltpu.SemaphoreType.DMA,
        ],
    )(_sc_body)
    return f(features, ids)


def kernel(features, coords):
    coords = lax.stop_gradient(coords)
    norm, ids3 = _coords_call(coords)
    ids = ids3.reshape(_B, _N)
    vox, _ = _sc_call(features, ids)
    return vox.reshape(_B, _C, _R, _R, _R), norm


# speedup vs baseline: 1.9416x; 1.0377x over previous
import functools

import jax
import jax.numpy as jnp
from jax import lax
from jax.experimental import pallas as pl
from jax.experimental.pallas import tpu as pltpu
from jax.experimental.pallas import tpu_sc as plsc

_R = 32
_B = 8
_C = 64
_N = 65536
_V = _R * _R * _R
_P = 8192


def _coords_body(coords_ref, norm_ref, ids_ref):
    c = coords_ref[...]
    mean = jnp.mean(c, axis=2, keepdims=True)
    n = (c - mean + 1.0) / 2.0
    scaled = jnp.clip(n * float(_R), 0.0, float(_R - 1))
    norm_ref[...] = scaled
    v = jnp.round(scaled).astype(jnp.int32)
    ids_ref[...] = (v[:, 0:1, :] * (_R * _R) + v[:, 1:2, :] * _R
                    + v[:, 2:3, :])


def _coords_call(coords):
    return pl.pallas_call(
        _coords_body,
        grid=(_B,),
        in_specs=[pl.BlockSpec((1, 3, _N), lambda b: (b, 0, 0))],
        out_specs=[
            pl.BlockSpec((1, 3, _N), lambda b: (b, 0, 0)),
            pl.BlockSpec((1, 1, _N), lambda b: (b, 0, 0)),
        ],
        out_shape=[
            jax.ShapeDtypeStruct((_B, 3, _N), jnp.float32),
            jax.ShapeDtypeStruct((_B, 1, _N), jnp.int32),
        ],
    )(coords)


def _sc_body(feat_hbm, ids_hbm, out_hbm, inv_hbm,
             ids_v, acc_v, buf0, buf1, sem0, sem1, osem):
    cid = lax.axis_index("c")
    sid = lax.axis_index("s")
    wid = cid * 16 + sid
    b = wid // 4
    q = wid % 4
    c0 = q * 16
    zero16 = jnp.zeros((16,), jnp.float32)
    one16 = jnp.full((16,), 1.0, jnp.float32)
    _NCH = _N // (2 * _P)
    _NIV = _V // (2 * _P)

    pltpu.async_copy(feat_hbm.at[b, c0, pl.ds(0, _P)], buf0, sem0)
    pltpu.async_copy(feat_hbm.at[b, c0, pl.ds(_P, _P)], buf1, sem1)

    pltpu.sync_copy(ids_hbm.at[b], ids_v)

    def zero_all_acc():
        @plsc.parallel_loop(0, _V // 16, 1, unroll=8)
        def _(j):
            acc_v[pl.ds(j * 16, 16)] = zero16

    zero_all_acc()

    @plsc.parallel_loop(0, _N // 16, 1, unroll=16)
    def _(i):
        plsc.addupdate_scatter(acc_v, [ids_v[pl.ds(i * 16, 16)]], one16)

    @plsc.parallel_loop(0, _V // 16, 1, unroll=4)
    def _(j):
        s = pl.ds(j * 16, 16)
        acc_v[s] = 1.0 / jnp.maximum(acc_v[s], 1.0)

    pltpu.sync_copy(acc_v, inv_hbm.at[b, q])

    def scatter_chunk(buf, base):
        @plsc.parallel_loop(0, _P // 16, 1, unroll=16)
        def _(i):
            plsc.addupdate_scatter(
                acc_v, [ids_v[pl.ds(base + i * 16, 16)]],
                buf[pl.ds(i * 16, 16)])

    def chan(ci, _):
        c = c0 + ci
        zero_all_acc()

        def pair(k2, _):
            base = k2 * (2 * _P)
            pltpu.make_async_copy(
                feat_hbm.at[b, c, pl.ds(0, _P)], buf0, sem0).wait()
            scatter_chunk(buf0, base)

            @pl.when(k2 < _NCH - 1)
            def _():
                pltpu.async_copy(
                    feat_hbm.at[b, c, pl.ds(base + 2 * _P, _P)], buf0, sem0)
            pltpu.make_async_copy(
                feat_hbm.at[b, c, pl.ds(0, _P)], buf1, sem1).wait()
            scatter_chunk(buf1, base + _P)

            @pl.when(k2 < _NCH - 1)
            def _():
                pltpu.async_copy(
                    feat_hbm.at[b, c, pl.ds(base + 3 * _P, _P)], buf1, sem1)
            return 0
        lax.fori_loop(0, _NCH, pair, 0)

        pltpu.async_copy(inv_hbm.at[b, q, pl.ds(0, _P)], buf0, sem0)
        pltpu.async_copy(inv_hbm.at[b, q, pl.ds(_P, _P)], buf1, sem1)

        def scale_pair(j2, _):
            base = j2 * (2 * _P)
            pltpu.make_async_copy(
                inv_hbm.at[b, q, pl.ds(0, _P)], buf0, sem0).wait()

            @plsc.parallel_loop(0, _P // 16, 1, unroll=8)
            def _(i):
                s = pl.ds(base + i * 16, 16)
                acc_v[s] = acc_v[s] * buf0[pl.ds(i * 16, 16)]

            @pl.when(j2 < _NIV - 1)
            def _():
                pltpu.async_copy(
                    inv_hbm.at[b, q, pl.ds(base + 2 * _P, _P)], buf0, sem0)
            pltpu.make_async_copy(
                inv_hbm.at[b, q, pl.ds(0, _P)], buf1, sem1).wait()

            @plsc.parallel_loop(0, _P // 16, 1, unroll=8)
            def _(i):
                s = pl.ds(base + _P + i * 16, 16)
                acc_v[s] = acc_v[s] * buf1[pl.ds(i * 16, 16)]

            @pl.when(j2 < _NIV - 1)
            def _():
                pltpu.async_copy(
                    inv_hbm.at[b, q, pl.ds(base + 3 * _P, _P)], buf1, sem1)
            return 0
        lax.fori_loop(0, _NIV, scale_pair, 0)

        pltpu.async_copy(acc_v, out_hbm.at[b, c], osem)

        @pl.when(ci < 15)
        def _():
            pltpu.async_copy(feat_hbm.at[b, c + 1, pl.ds(0, _P)], buf0, sem0)
            pltpu.async_copy(feat_hbm.at[b, c + 1, pl.ds(_P, _P)], buf1, sem1)
        pltpu.make_async_copy(acc_v, out_hbm.at[b, c], osem).wait()
        return 0
    lax.fori_loop(0, 16, chan, 0)


def _sc_call(features, ids):
    mesh = plsc.VectorSubcoreMesh(core_axis_name="c", subcore_axis_name="s")
    f = functools.partial(
        pl.kernel,
        out_type=[
            jax.ShapeDtypeStruct((_B, _C, _V), jnp.float32),
            jax.ShapeDtypeStruct((_B, 4, _V), jnp.float32),
        ],
        mesh=mesh,
        compiler_params=pltpu.CompilerParams(needs_layout_passes=False),
        scratch_types=[
            pltpu.VMEM((_N,), jnp.int32),
            pltpu.VMEM((_V,), jnp.float32),
            pltpu.VMEM((_P,), jnp.float32),
            pltpu.VMEM((_P,), jnp.float32),
            pltpu.SemaphoreType.DMA,
            pltpu.SemaphoreType.DMA,
            pltpu.SemaphoreType.DMA,
        ],
    )(_sc_body)
    return f(features, ids)


def kernel(features, coords):
    coords = lax.stop_gradient(coords)
    norm, ids3 = _coords_call(coords)
    ids = ids3.reshape(_B, _N)
    vox, _ = _sc_call(features, ids)
    return vox.reshape(_B, _C, _R, _R, _R), norm
